# baseline (speedup 1.0000x reference)
import jax
import jax.numpy as jnp
from jax import lax
from jax.experimental import pallas as pl
from jax.experimental.pallas import tpu as pltpu
from jax.experimental.pallas import tpu_sc as plsc

N = 10000
E = 320000
H = 128
EF = 4
NG = 16

NC = 2
NS = 16
NW = NC * NS

C = 100
NCHUNK = E // C
CPW = NCHUNK // NW
ROWS_PER_TILE = N // NS

EB = 3200
NB = 2000

_F32 = jnp.float32


def _silu(v):
    return v * jax.nn.sigmoid(v)


def _node_proj_body(h_ref, whi_ref, whj_ref, be1_ref, a_ref, b_ref):
    hv = h_ref[...]
    a_ref[...] = (
        jnp.dot(hv, whi_ref[...], preferred_element_type=_F32) + be1_ref[...]
    ).astype(jnp.bfloat16)
    b_ref[...] = jnp.dot(
        hv, whj_ref[...], preferred_element_type=_F32).astype(jnp.bfloat16)


def _node_proj(h, whi, whj, be1):
    grid = N // NB
    return pl.pallas_call(
        _node_proj_body,
        grid=(grid,),
        in_specs=[
            pl.BlockSpec((NB, H), lambda i: (i, 0)),
            pl.BlockSpec((H, H), lambda i: (0, 0)),
            pl.BlockSpec((H, H), lambda i: (0, 0)),
            pl.BlockSpec((1, H), lambda i: (0, 0)),
        ],
        out_specs=[
            pl.BlockSpec((NB, H), lambda i: (i, 0)),
            pl.BlockSpec((NB, H), lambda i: (i, 0)),
        ],
        out_shape=[
            jax.ShapeDtypeStruct((N, H), jnp.bfloat16),
            jax.ShapeDtypeStruct((N, H), jnp.bfloat16),
        ],
    )(h, whi, whj, be1)


def _sc_gather_body(cpw, a_hbm, b_hbm, x16_hbm, dst_hbm, src_hbm,
                    pre_a_hbm, pre_b_hbm, rel_hbm,
                    idxd, idxs, buf_a, buf_b, buf_xt, buf_xs, buf_rel,
                    ga, gb, gxt, gxs, wsem):
    cid = lax.axis_index("c")
    sid = lax.axis_index("s")
    wid = sid * NC + cid
    base = wid * cpw

    def do_chunk(j, k):
        c = base + 2 * j + k

        @pl.when(j > 0)
        def _():
            pltpu.make_async_copy(
                buf_a[k], pre_a_hbm.at[pl.ds(c * C, C)], wsem[k]).wait()
            pltpu.make_async_copy(
                buf_b[k], pre_b_hbm.at[pl.ds(c * C, C)], wsem[k]).wait()
            pltpu.make_async_copy(
                buf_rel[k], rel_hbm.at[pl.ds(c * C * 16, C * 16)],
                wsem[k]).wait()

        pltpu.sync_copy(dst_hbm.at[c], idxd[k])
        pltpu.sync_copy(src_hbm.at[c], idxs[k])
        cp_a = pltpu.async_copy(a_hbm.at[idxd[k]], buf_a[k], ga)
        cp_b = pltpu.async_copy(b_hbm.at[idxs[k]], buf_b[k], gb)
        cp_xt = pltpu.async_copy(x16_hbm.at[idxd[k]], buf_xt[k], gxt)
        cp_xs = pltpu.async_copy(x16_hbm.at[idxs[k]], buf_xs[k], gxs)
        cp_a.wait()
        cp_b.wait()
        cp_xt.wait()
        cp_xs.wait()
        for j2 in range(C):
            buf_rel[k][pl.ds(j2 * 16, 16)] = (
                buf_xt[k][j2, :] - buf_xs[k][j2, :])
        pltpu.async_copy(buf_a[k], pre_a_hbm.at[pl.ds(c * C, C)], wsem[k])
        pltpu.async_copy(buf_b[k], pre_b_hbm.at[pl.ds(c * C, C)], wsem[k])
        pltpu.async_copy(
            buf_rel[k], rel_hbm.at[pl.ds(c * C * 16, C * 16)], wsem[k])

    def body(j, carry):
        do_chunk(j, 0)
        do_chunk(j, 1)
        return carry

    lax.fori_loop(0, cpw // 2, body, 0)
    for k in range(2):
        pltpu.make_async_copy(
            buf_a[k], pre_a_hbm.at[pl.ds(base * C, C)], wsem[k]).wait()
        pltpu.make_async_copy(
            buf_b[k], pre_b_hbm.at[pl.ds(base * C, C)], wsem[k]).wait()
        pltpu.make_async_copy(
            buf_rel[k], rel_hbm.at[pl.ds(base * C * 16, C * 16)],
            wsem[k]).wait()


def _sc_gather(a, b, x16, dst2d, src2d):
    import functools
    nchunk = dst2d.shape[0]
    e2 = nchunk * C
    mesh = plsc.VectorSubcoreMesh(
        core_axis_name="c", subcore_axis_name="s",
        num_cores=NC, num_subcores=NS)
    fn = pl.kernel(
        functools.partial(_sc_gather_body, nchunk // NW),
        out_type=[
            jax.ShapeDtypeStruct((e2, H), jnp.bfloat16),
            jax.ShapeDtypeStruct((e2, H), jnp.bfloat16),
            jax.ShapeDtypeStruct((e2 * 16,), _F32),
        ],
        mesh=mesh,
        scratch_types=[
            [pltpu.VMEM((C,), jnp.int32)] * 2,
            [pltpu.VMEM((C,), jnp.int32)] * 2,
            [pltpu.VMEM((C, H), jnp.bfloat16)] * 2,
            [pltpu.VMEM((C, H), jnp.bfloat16)] * 2,
            [pltpu.VMEM((C, 16), _F32)] * 2,
            [pltpu.VMEM((C, 16), _F32)] * 2,
            [pltpu.VMEM((C * 16,), _F32)] * 2,
            pltpu.SemaphoreType.DMA,
            pltpu.SemaphoreType.DMA,
            pltpu.SemaphoreType.DMA,
            pltpu.SemaphoreType.DMA,
            [pltpu.SemaphoreType.DMA] * 2,
        ],
        compiler_params=pltpu.CompilerParams(use_tc_tiling_on_sc=False),
    )
    return fn(a, b, x16, dst2d, src2d)


def _edge_mlp_body(pre_a_ref, pre_b_ref, relp_ref, attrt_ref, mtile_ref,
                   wd_ref, wa_ref, we2_ref, be2_ref, wei_ref,
                   bei_ref, wx1_ref, bx1_ref, wx2_ref,
                   msg_ref, w_ref):
    pre = (pre_a_ref[...].astype(_F32)
           + pre_b_ref[...].astype(_F32))
    relp = relp_ref[...]
    rep = jnp.broadcast_to(
        relp[:, None, :], (EB // 8, 8, 128)).reshape(EB, 128)
    m = jnp.broadcast_to(
        mtile_ref[...][None], (EB // 8, 8, 128)).reshape(EB, 128)
    rel_ext = rep * m
    ones_col = jnp.ones((128, 1), _F32)
    d_sq = jnp.dot(rel_ext * rel_ext, ones_col,
                   preferred_element_type=_F32)
    d16 = jnp.sqrt(jnp.broadcast_to(d_sq, (EB, 16)) + 1e-08)
    step = 10.0 / (NG - 1)
    offs = lax.broadcasted_iota(jnp.int32, (1, NG), 1).astype(_F32) * step
    coeff = -0.5 / step**2
    d_feat = jnp.exp(coeff * (d16 - offs) ** 2)
    pre = pre + jnp.dot(d_feat, wd_ref[...], preferred_element_type=_F32)
    pre = pre + lax.dot_general(
        attrt_ref[...], wa_ref[...],
        dimension_numbers=(((0,), (0,)), ((), ())),
        preferred_element_type=_F32)
    t = _silu(pre)
    mij = _silu(
        jnp.dot(t, we2_ref[...], preferred_element_type=_F32) + be2_ref[...])
    e16 = jnp.broadcast_to(
        jnp.dot(mij, wei_ref[...], preferred_element_type=_F32)
        + bei_ref[...], (EB, 16))
    eij = jax.nn.sigmoid(e16)[:, :1]
    msg_ref[...] = mij * eij
    g = _silu(
        jnp.dot(mij, wx1_ref[...], preferred_element_type=_F32) + bx1_ref[...])
    s16 = jnp.broadcast_to(
        jnp.dot(g, wx2_ref[...], preferred_element_type=_F32), (EB, 16))
    w_ref[...] = jnp.tanh(s16) / (d16 + 1.0)


def _edge_mlp(pre_a, pre_b, relpack, attrt, mtile,
              wd, wa, we2, be2, wei, bei, wx1, bx1, wx2, off):
    e2 = pre_a.shape[0]
    grid = e2 // EB
    full = lambda r, c: pl.BlockSpec((r, c), lambda i: (0, 0))
    return pl.pallas_call(
        _edge_mlp_body,
        grid=(grid,),
        in_specs=[
            pl.BlockSpec((EB, H), lambda i: (i, 0)),
            pl.BlockSpec((EB, H), lambda i: (i, 0)),
            pl.BlockSpec((EB * 16 // 128, 128), lambda i: (i, 0)),
            pl.BlockSpec((EF, EB), lambda i: (0, i + off)),
            full(8, 128),
            full(NG, H), full(EF, H), full(H, H), full(1, H),
            full(H, 1), full(1, 1), full(H, H), full(1, H), full(H, 1),
        ],
        out_specs=[
            pl.BlockSpec((EB, H), lambda i: (i, 0)),
            pl.BlockSpec((EB, 16), lambda i: (i, 0)),
        ],
        out_shape=[
            jax.ShapeDtypeStruct((e2, H), _F32),
            jax.ShapeDtypeStruct((e2, 16), _F32),
        ],
    )(pre_a, pre_b, relpack, attrt, mtile,
      wd, wa, we2, be2, wei, bei, wx1, bx1, wx2)


def _sc_scatter_body(cpw, msg_hbm, w_hbm, rel_hbm, dst_hbm, zmi_hbm, zdx_hbm,
                     mi2_hbm, dx2_hbm,
                     mbuf, wbuf, rbuf, dbuf, idxd, acc_mi, acc_dx,
                     lm, lw, lr, asem):
    cid = lax.axis_index("c")
    sid = lax.axis_index("s")
    wid = sid * NC + cid
    base = wid * cpw
    r0 = sid * ROWS_PER_TILE

    pltpu.sync_copy(zmi_hbm.at[pl.ds(r0, ROWS_PER_TILE)],
                    acc_mi.at[pl.ds(r0, ROWS_PER_TILE)])
    pltpu.sync_copy(zdx_hbm.at[pl.ds(r0, ROWS_PER_TILE)],
                    acc_dx.at[pl.ds(r0, ROWS_PER_TILE)])
    plsc.subcore_barrier()

    def do_chunk(j, k):
        c = base + 2 * j + k

        @pl.when(j > 0)
        def _():
            pltpu.make_async_copy(mbuf[k], acc_mi.at[idxd[k]], asem[k]).wait()
            pltpu.make_async_copy(dbuf[k], acc_dx.at[idxd[k]], asem[k]).wait()

        pltpu.sync_copy(dst_hbm.at[c], idxd[k])
        cp_m = pltpu.async_copy(msg_hbm.at[pl.ds(c * C, C)], mbuf[k], lm)
        cp_w = pltpu.async_copy(w_hbm.at[pl.ds(c * C, C)], wbuf[k], lw)
        cp_r = pltpu.async_copy(
            rel_hbm.at[pl.ds(c * C * 16, C * 16)], rbuf[k], lr)
        cp_m.wait()
        cp_w.wait()
        cp_r.wait()
        for j2 in range(C):
            dbuf[k][j2, :] = rbuf[k][pl.ds(j2 * 16, 16)] * wbuf[k][j2, :]
        pltpu.async_copy(mbuf[k], acc_mi.at[idxd[k]], asem[k], add=True)
        pltpu.async_copy(dbuf[k], acc_dx.at[idxd[k]], asem[k], add=True)

    def body(j, carry):
        do_chunk(j, 0)
        do_chunk(j, 1)
        return carry

    lax.fori_loop(0, cpw // 2, body, 0)
    for k in range(2):
        pltpu.make_async_copy(mbuf[k], acc_mi.at[idxd[k]], asem[k]).wait()
        pltpu.make_async_copy(dbuf[k], acc_dx.at[idxd[k]], asem[k]).wait()
    plsc.subcore_barrier()

    pltpu.sync_copy(acc_mi.at[pl.ds(r0, ROWS_PER_TILE)],
                    mi2_hbm.at[cid, pl.ds(r0, ROWS_PER_TILE)])
    pltpu.sync_copy(acc_dx.at[pl.ds(r0, ROWS_PER_TILE)],
                    dx2_hbm.at[cid, pl.ds(r0, ROWS_PER_TILE)])


def _sc_scatter(msg, w, rel1d, dst2d, zmi, zdx):
    import functools
    nchunk = dst2d.shape[0]
    mesh = plsc.VectorSubcoreMesh(
        core_axis_name="c", subcore_axis_name="s",
        num_cores=NC, num_subcores=NS)
    fn = pl.kernel(
        functools.partial(_sc_scatter_body, nchunk // NW),
        out_type=[
            jax.ShapeDtypeStruct((NC, N, H), _F32),
            jax.ShapeDtypeStruct((NC, N, 16), _F32),
        ],
        mesh=mesh,
        scratch_types=[
            [pltpu.VMEM((C, H), _F32)] * 2,
            [pltpu.VMEM((C, 16), _F32)] * 2,
            [pltpu.VMEM((C * 16,), _F32)] * 2,
            [pltpu.VMEM((C, 16), _F32)] * 2,
            [pltpu.VMEM((C,), jnp.int32)] * 2,
            pltpu.VMEM_SHARED((N, H), _F32),
            pltpu.VMEM_SHARED((N, 16), _F32),
            pltpu.SemaphoreType.DMA,
            pltpu.SemaphoreType.DMA,
            pltpu.SemaphoreType.DMA,
            [pltpu.SemaphoreType.DMA] * 2,
        ],
        compiler_params=pltpu.CompilerParams(use_tc_tiling_on_sc=False),
    )
    return fn(msg, w, rel1d, dst2d, zmi, zdx)


def _node_upd_body(h_ref, x_ref, mask_ref, mi0_ref, mi1_ref, mi2_ref, mi3_ref,
                   dx0_ref, dx1_ref, dx2_ref, dx3_ref,
                   wm_ref, wh_ref, bn1_ref, wn2_ref, bn2_ref,
                   ho_ref, xo_ref):
    hv = h_ref[...]
    mi = (mi0_ref[...] + mi1_ref[...]) + (mi2_ref[...] + mi3_ref[...])
    t = _silu(
        jnp.dot(mi, wm_ref[...], preferred_element_type=_F32)
        + jnp.dot(hv, wh_ref[...], preferred_element_type=_F32)
        + bn1_ref[...])
    dh = jnp.dot(t, wn2_ref[...], preferred_element_type=_F32) + bn2_ref[...]
    ho_ref[...] = hv + dh
    dx = (dx0_ref[...] + dx1_ref[...]) + (dx2_ref[...] + dx3_ref[...])
    xo_ref[...] = x_ref[...] + dx[:, :3] * mask_ref[...]


def _node_upd(h, x, mask, mis, dxs, wm, wh, bn1, wn2, bn2):
    grid = N // NB
    full = lambda r, c: pl.BlockSpec((r, c), lambda i: (0, 0))
    return pl.pallas_call(
        _node_upd_body,
        grid=(grid,),
        in_specs=[
            pl.BlockSpec((NB, H), lambda i: (i, 0)),
            pl.BlockSpec((NB, 3), lambda i: (i, 0)),
            pl.BlockSpec((NB, 1), lambda i: (i, 0)),
            pl.BlockSpec((NB, H), lambda i: (i, 0)),
            pl.BlockSpec((NB, H), lambda i: (i, 0)),
            pl.BlockSpec((NB, H), lambda i: (i, 0)),
            pl.BlockSpec((NB, H), lambda i: (i, 0)),
            pl.BlockSpec((NB, 16), lambda i: (i, 0)),
            pl.BlockSpec((NB, 16), lambda i: (i, 0)),
            pl.BlockSpec((NB, 16), lambda i: (i, 0)),
            pl.BlockSpec((NB, 16), lambda i: (i, 0)),
            full(H, H), full(H, H), full(1, H), full(H, H), full(1, H),
        ],
        out_specs=[
            pl.BlockSpec((NB, H), lambda i: (i, 0)),
            pl.BlockSpec((NB, 3), lambda i: (i, 0)),
        ],
        out_shape=[
            jax.ShapeDtypeStruct((N, H), _F32),
            jax.ShapeDtypeStruct((N, 3), _F32),
        ],
    )(h, x, mask, *mis, *dxs, wm, wh, bn1, wn2, bn2)


def kernel(h, x, edge_index, mask_ligand, edge_attr,
           We1, be1, We2, be2, Wei, bei, Wx1, bx1, Wx2,
           Wn1, bn1, Wn2, bn2):
    src = edge_index[0]
    dst = edge_index[1]
    dst2d = dst.reshape(NCHUNK, C)
    src2d = src.reshape(NCHUNK, C)
    x16 = jnp.pad(x, ((0, 0), (0, 13)))

    a, b = _node_proj(h, We1[:H], We1[H:2 * H], be1.reshape(1, H))

    mtile = (jnp.arange(128)[None, :] // 16
             == jnp.arange(8)[:, None]).astype(_F32)
    attrt = edge_attr.T
    zmi = jnp.zeros((N, H), _F32)
    zdx = jnp.zeros((N, 16), _F32)

    nh = NCHUNK // 2
    gathered = []
    for hx in range(2):
        sl = slice(hx * nh, (hx + 1) * nh)
        gathered.append(_sc_gather(a, b, x16, dst2d[sl], src2d[sl]))
    mis, dxs = [], []
    for hx in range(2):
        pre_a, pre_b, rel1d = gathered[hx]
        e2 = pre_a.shape[0]
        relpack = rel1d.reshape(e2 * 16 // 128, 128)
        msg, w = _edge_mlp(
            pre_a, pre_b, relpack, attrt, mtile,
            We1[2 * H:2 * H + NG], We1[2 * H + NG:],
            We2, be2.reshape(1, H), Wei, bei.reshape(1, 1),
            Wx1, bx1.reshape(1, H), Wx2, hx * (e2 // EB))
        mi2, dx2 = _sc_scatter(msg, w, rel1d, dst2d[hx * nh:(hx + 1) * nh],
                               zmi, zdx)
        mis.extend([mi2[0], mi2[1]])
        dxs.extend([dx2[0], dx2[1]])

    h_out, x_out = _node_upd(
        h, x, mask_ligand.reshape(N, 1), mis, dxs,
        Wn1[:H], Wn1[H:], bn1.reshape(1, H), Wn2, bn2.reshape(1, H))
    return (h_out, x_out)


# speedup vs baseline: 1.7684x; 1.7684x over previous
import jax
import jax.numpy as jnp
from jax import lax
from jax.experimental import pallas as pl
from jax.experimental.pallas import tpu as pltpu
from jax.experimental.pallas import tpu_sc as plsc

N = 10000
E = 320000
H = 128
EF = 4
NG = 16

NC = 2
NS = 16
NW = NC * NS

C = 100
NCHUNK = E // C
CPW = NCHUNK // NW
ROWS_PER_TILE = N // NS

EB = 3200
NB = 2000

_F32 = jnp.float32


def _silu(v):
    return v * jax.nn.sigmoid(v)


def _node_proj_body(h_ref, whi_ref, whj_ref, be1_ref, a_ref, b_ref):
    hv = h_ref[...]
    a_ref[...] = (
        jnp.dot(hv, whi_ref[...], preferred_element_type=_F32) + be1_ref[...]
    )
    b_ref[...] = jnp.dot(hv, whj_ref[...], preferred_element_type=_F32)


def _node_proj(h, whi, whj, be1):
    grid = N // NB
    return pl.pallas_call(
        _node_proj_body,
        grid=(grid,),
        in_specs=[
            pl.BlockSpec((NB, H), lambda i: (i, 0)),
            pl.BlockSpec((H, H), lambda i: (0, 0)),
            pl.BlockSpec((H, H), lambda i: (0, 0)),
            pl.BlockSpec((1, H), lambda i: (0, 0)),
        ],
        out_specs=[
            pl.BlockSpec((NB, H), lambda i: (i, 0)),
            pl.BlockSpec((NB, H), lambda i: (i, 0)),
        ],
        out_shape=[
            jax.ShapeDtypeStruct((N, H), _F32),
            jax.ShapeDtypeStruct((N, H), _F32),
        ],
    )(h, whi, whj, be1)


def _sc_gather_body(cpw, a_hbm, b_hbm, x16_hbm, dst_hbm, src_hbm,
                    pre_a_hbm, pre_b_hbm, rel_hbm,
                    idxd, idxs, buf_a, buf_b, buf_xt, buf_xs, buf_rel,
                    ga, gb, gxt, gxs, wsem, isem):
    cid = lax.axis_index("c")
    sid = lax.axis_index("s")
    wid = sid * NC + cid
    base = wid * cpw

    pltpu.sync_copy(dst_hbm.at[base], idxd[0])
    pltpu.sync_copy(src_hbm.at[base], idxs[0])

    def do_chunk(j, k):
        c = base + 2 * j + k

        @pl.when(c > base)
        def _():
            pltpu.make_async_copy(dst_hbm.at[c], idxd[k], isem[k]).wait()
            pltpu.make_async_copy(src_hbm.at[c], idxs[k], isem[k]).wait()

        @pl.when(j > 0)
        def _():
            pltpu.make_async_copy(
                buf_a[k], pre_a_hbm.at[pl.ds(c * C, C)], wsem[k]).wait()
            pltpu.make_async_copy(
                buf_b[k], pre_b_hbm.at[pl.ds(c * C, C)], wsem[k]).wait()
            pltpu.make_async_copy(
                buf_rel[k], rel_hbm.at[pl.ds(c * C * 16, C * 16)],
                wsem[k]).wait()

        @pl.when(c + 1 < base + cpw)
        def _():
            pltpu.async_copy(dst_hbm.at[c + 1], idxd[1 - k], isem[1 - k])
            pltpu.async_copy(src_hbm.at[c + 1], idxs[1 - k], isem[1 - k])

        cp_a = pltpu.async_copy(a_hbm.at[idxd[k]], buf_a[k], ga)
        cp_b = pltpu.async_copy(b_hbm.at[idxs[k]], buf_b[k], gb)
        cp_xt = pltpu.async_copy(x16_hbm.at[idxd[k]], buf_xt[k], gxt)
        cp_xs = pltpu.async_copy(x16_hbm.at[idxs[k]], buf_xs[k], gxs)
        cp_a.wait()
        cp_b.wait()
        cp_xt.wait()
        cp_xs.wait()
        for j2 in range(C):
            buf_rel[k][pl.ds(j2 * 16, 16)] = (
                buf_xt[k][j2, :] - buf_xs[k][j2, :])
        pltpu.async_copy(buf_a[k], pre_a_hbm.at[pl.ds(c * C, C)], wsem[k])
        pltpu.async_copy(buf_b[k], pre_b_hbm.at[pl.ds(c * C, C)], wsem[k])
        pltpu.async_copy(
            buf_rel[k], rel_hbm.at[pl.ds(c * C * 16, C * 16)], wsem[k])

    def body(j, carry):
        do_chunk(j, 0)
        do_chunk(j, 1)
        return carry

    lax.fori_loop(0, cpw // 2, body, 0)
    for k in range(2):
        pltpu.make_async_copy(
            buf_a[k], pre_a_hbm.at[pl.ds(base * C, C)], wsem[k]).wait()
        pltpu.make_async_copy(
            buf_b[k], pre_b_hbm.at[pl.ds(base * C, C)], wsem[k]).wait()
        pltpu.make_async_copy(
            buf_rel[k], rel_hbm.at[pl.ds(base * C * 16, C * 16)],
            wsem[k]).wait()


def _sc_gather(a, b, x16, dst2d, src2d):
    import functools
    nchunk = dst2d.shape[0]
    e2 = nchunk * C
    mesh = plsc.VectorSubcoreMesh(
        core_axis_name="c", subcore_axis_name="s",
        num_cores=NC, num_subcores=NS)
    fn = pl.kernel(
        functools.partial(_sc_gather_body, nchunk // NW),
        out_type=[
            jax.ShapeDtypeStruct((e2, H), _F32),
            jax.ShapeDtypeStruct((e2, H), _F32),
            jax.ShapeDtypeStruct((e2 * 16,), _F32),
        ],
        mesh=mesh,
        scratch_types=[
            [pltpu.VMEM((C,), jnp.int32)] * 2,
            [pltpu.VMEM((C,), jnp.int32)] * 2,
            [pltpu.VMEM((C, H), _F32)] * 2,
            [pltpu.VMEM((C, H), _F32)] * 2,
            [pltpu.VMEM((C, 16), _F32)] * 2,
            [pltpu.VMEM((C, 16), _F32)] * 2,
            [pltpu.VMEM((C * 16,), _F32)] * 2,
            pltpu.SemaphoreType.DMA,
            pltpu.SemaphoreType.DMA,
            pltpu.SemaphoreType.DMA,
            pltpu.SemaphoreType.DMA,
            [pltpu.SemaphoreType.DMA] * 2,
            [pltpu.SemaphoreType.DMA] * 2,
        ],
        compiler_params=pltpu.CompilerParams(use_tc_tiling_on_sc=False),
    )
    return fn(a, b, x16, dst2d, src2d)


def _edge_mlp_body(pre_a_ref, pre_b_ref, relp_ref, attrt_ref, mtile_ref,
                   wd_ref, wa_ref, we2_ref, be2_ref, wei_ref,
                   bei_ref, wx1_ref, bx1_ref, wx2_ref,
                   msg_ref, w_ref):
    pre = pre_a_ref[...] + pre_b_ref[...]
    relp = relp_ref[...]
    rep = jnp.broadcast_to(
        relp[:, None, :], (EB // 8, 8, 128)).reshape(EB, 128)
    m = jnp.broadcast_to(
        mtile_ref[...][None], (EB // 8, 8, 128)).reshape(EB, 128)
    rel_ext = rep * m
    ones_col = jnp.ones((128, 1), _F32)
    d_sq = jnp.dot(rel_ext * rel_ext, ones_col,
                   preferred_element_type=_F32)
    d16 = jnp.sqrt(jnp.broadcast_to(d_sq, (EB, 16)) + 1e-08)
    step = 10.0 / (NG - 1)
    offs = lax.broadcasted_iota(jnp.int32, (1, NG), 1).astype(_F32) * step
    coeff = -0.5 / step**2
    d_feat = jnp.exp(coeff * (d16 - offs) ** 2)
    pre = pre + jnp.dot(d_feat, wd_ref[...], preferred_element_type=_F32)
    pre = pre + lax.dot_general(
        attrt_ref[...], wa_ref[...],
        dimension_numbers=(((0,), (0,)), ((), ())),
        preferred_element_type=_F32)
    t = _silu(pre)
    mij = _silu(
        jnp.dot(t, we2_ref[...], preferred_element_type=_F32) + be2_ref[...])
    e16 = jnp.broadcast_to(
        jnp.dot(mij, wei_ref[...], preferred_element_type=_F32)
        + bei_ref[...], (EB, 16))
    eij = jax.nn.sigmoid(e16)[:, :1]
    msg_ref[...] = mij * eij
    g = _silu(
        jnp.dot(mij, wx1_ref[...], preferred_element_type=_F32) + bx1_ref[...])
    s16 = jnp.broadcast_to(
        jnp.dot(g, wx2_ref[...], preferred_element_type=_F32), (EB, 16))
    w_ref[...] = jnp.tanh(s16) / (d16 + 1.0)


def _edge_mlp(pre_a, pre_b, relpack, attrt, mtile,
              wd, wa, we2, be2, wei, bei, wx1, bx1, wx2, off):
    e2 = pre_a.shape[0]
    grid = e2 // EB
    full = lambda r, c: pl.BlockSpec((r, c), lambda i: (0, 0))
    return pl.pallas_call(
        _edge_mlp_body,
        grid=(grid,),
        in_specs=[
            pl.BlockSpec((EB, H), lambda i: (i, 0)),
            pl.BlockSpec((EB, H), lambda i: (i, 0)),
            pl.BlockSpec((EB * 16 // 128, 128), lambda i: (i, 0)),
            pl.BlockSpec((EF, EB), lambda i: (0, i + off)),
            full(8, 128),
            full(NG, H), full(EF, H), full(H, H), full(1, H),
            full(H, 1), full(1, 1), full(H, H), full(1, H), full(H, 1),
        ],
        out_specs=[
            pl.BlockSpec((EB, H), lambda i: (i, 0)),
            pl.BlockSpec((EB, 16), lambda i: (i, 0)),
        ],
        out_shape=[
            jax.ShapeDtypeStruct((e2, H), _F32),
            jax.ShapeDtypeStruct((e2, 16), _F32),
        ],
    )(pre_a, pre_b, relpack, attrt, mtile,
      wd, wa, we2, be2, wei, bei, wx1, bx1, wx2)


def _sc_scatter_body(cpw, msg_hbm, w_hbm, rel_hbm, dst_hbm, zmi_hbm, zdx_hbm,
                     mi2_hbm, dx2_hbm,
                     mbuf, wbuf, rbuf, dbuf, idxd, acc_mi, acc_dx,
                     lm, lw, lr, asem):
    cid = lax.axis_index("c")
    sid = lax.axis_index("s")
    wid = sid * NC + cid
    base = wid * cpw
    r0 = sid * ROWS_PER_TILE

    pltpu.sync_copy(zmi_hbm.at[pl.ds(r0, ROWS_PER_TILE)],
                    acc_mi.at[pl.ds(r0, ROWS_PER_TILE)])
    pltpu.sync_copy(zdx_hbm.at[pl.ds(r0, ROWS_PER_TILE)],
                    acc_dx.at[pl.ds(r0, ROWS_PER_TILE)])
    plsc.subcore_barrier()

    def do_chunk(j, k):
        c = base + 2 * j + k

        @pl.when(j > 0)
        def _():
            pltpu.make_async_copy(mbuf[k], acc_mi.at[idxd[k]], asem[k]).wait()
            pltpu.make_async_copy(dbuf[k], acc_dx.at[idxd[k]], asem[k]).wait()

        pltpu.sync_copy(dst_hbm.at[c], idxd[k])
        cp_m = pltpu.async_copy(msg_hbm.at[pl.ds(c * C, C)], mbuf[k], lm)
        cp_w = pltpu.async_copy(w_hbm.at[pl.ds(c * C, C)], wbuf[k], lw)
        cp_r = pltpu.async_copy(
            rel_hbm.at[pl.ds(c * C * 16, C * 16)], rbuf[k], lr)
        cp_m.wait()
        cp_w.wait()
        cp_r.wait()
        for j2 in range(C):
            dbuf[k][j2, :] = rbuf[k][pl.ds(j2 * 16, 16)] * wbuf[k][j2, :]
        pltpu.async_copy(mbuf[k], acc_mi.at[idxd[k]], asem[k], add=True)
        pltpu.async_copy(dbuf[k], acc_dx.at[idxd[k]], asem[k], add=True)

    def body(j, carry):
        do_chunk(j, 0)
        do_chunk(j, 1)
        return carry

    lax.fori_loop(0, cpw // 2, body, 0)
    for k in range(2):
        pltpu.make_async_copy(mbuf[k], acc_mi.at[idxd[k]], asem[k]).wait()
        pltpu.make_async_copy(dbuf[k], acc_dx.at[idxd[k]], asem[k]).wait()
    plsc.subcore_barrier()

    pltpu.sync_copy(acc_mi.at[pl.ds(r0, ROWS_PER_TILE)],
                    mi2_hbm.at[cid, pl.ds(r0, ROWS_PER_TILE)])
    pltpu.sync_copy(acc_dx.at[pl.ds(r0, ROWS_PER_TILE)],
                    dx2_hbm.at[cid, pl.ds(r0, ROWS_PER_TILE)])


def _sc_scatter(msg, w, rel1d, dst2d, zmi, zdx):
    import functools
    nchunk = dst2d.shape[0]
    mesh = plsc.VectorSubcoreMesh(
        core_axis_name="c", subcore_axis_name="s",
        num_cores=NC, num_subcores=NS)
    fn = pl.kernel(
        functools.partial(_sc_scatter_body, nchunk // NW),
        out_type=[
            jax.ShapeDtypeStruct((NC, N, H), _F32),
            jax.ShapeDtypeStruct((NC, N, 16), _F32),
        ],
        mesh=mesh,
        scratch_types=[
            [pltpu.VMEM((C, H), _F32)] * 2,
            [pltpu.VMEM((C, 16), _F32)] * 2,
            [pltpu.VMEM((C * 16,), _F32)] * 2,
            [pltpu.VMEM((C, 16), _F32)] * 2,
            [pltpu.VMEM((C,), jnp.int32)] * 2,
            pltpu.VMEM_SHARED((N, H), _F32),
            pltpu.VMEM_SHARED((N, 16), _F32),
            pltpu.SemaphoreType.DMA,
            pltpu.SemaphoreType.DMA,
            pltpu.SemaphoreType.DMA,
            [pltpu.SemaphoreType.DMA] * 2,
        ],
        compiler_params=pltpu.CompilerParams(use_tc_tiling_on_sc=False),
    )
    return fn(msg, w, rel1d, dst2d, zmi, zdx)


def _node_upd_body(h_ref, x_ref, mask_ref, mi0_ref, mi1_ref, mi2_ref, mi3_ref,
                   dx0_ref, dx1_ref, dx2_ref, dx3_ref,
                   wm_ref, wh_ref, bn1_ref, wn2_ref, bn2_ref,
                   ho_ref, xo_ref):
    hv = h_ref[...]
    mi = (mi0_ref[...] + mi1_ref[...]) + (mi2_ref[...] + mi3_ref[...])
    t = _silu(
        jnp.dot(mi, wm_ref[...], preferred_element_type=_F32)
        + jnp.dot(hv, wh_ref[...], preferred_element_type=_F32)
        + bn1_ref[...])
    dh = jnp.dot(t, wn2_ref[...], preferred_element_type=_F32) + bn2_ref[...]
    ho_ref[...] = hv + dh
    dx = (dx0_ref[...] + dx1_ref[...]) + (dx2_ref[...] + dx3_ref[...])
    xo_ref[...] = x_ref[...] + dx[:, :3] * mask_ref[...]


def _node_upd(h, x, mask, mis, dxs, wm, wh, bn1, wn2, bn2):
    grid = N // NB
    full = lambda r, c: pl.BlockSpec((r, c), lambda i: (0, 0))
    return pl.pallas_call(
        _node_upd_body,
        grid=(grid,),
        in_specs=[
            pl.BlockSpec((NB, H), lambda i: (i, 0)),
            pl.BlockSpec((NB, 3), lambda i: (i, 0)),
            pl.BlockSpec((NB, 1), lambda i: (i, 0)),
            pl.BlockSpec((NB, H), lambda i: (i, 0)),
            pl.BlockSpec((NB, H), lambda i: (i, 0)),
            pl.BlockSpec((NB, H), lambda i: (i, 0)),
            pl.BlockSpec((NB, H), lambda i: (i, 0)),
            pl.BlockSpec((NB, 16), lambda i: (i, 0)),
            pl.BlockSpec((NB, 16), lambda i: (i, 0)),
            pl.BlockSpec((NB, 16), lambda i: (i, 0)),
            pl.BlockSpec((NB, 16), lambda i: (i, 0)),
            full(H, H), full(H, H), full(1, H), full(H, H), full(1, H),
        ],
        out_specs=[
            pl.BlockSpec((NB, H), lambda i: (i, 0)),
            pl.BlockSpec((NB, 3), lambda i: (i, 0)),
        ],
        out_shape=[
            jax.ShapeDtypeStruct((N, H), _F32),
            jax.ShapeDtypeStruct((N, 3), _F32),
        ],
    )(h, x, mask, *mis, *dxs, wm, wh, bn1, wn2, bn2)


def kernel(h, x, edge_index, mask_ligand, edge_attr,
           We1, be1, We2, be2, Wei, bei, Wx1, bx1, Wx2,
           Wn1, bn1, Wn2, bn2):
    src = edge_index[0]
    dst = edge_index[1]
    dst2d = dst.reshape(NCHUNK, C)
    src2d = src.reshape(NCHUNK, C)
    x16 = jnp.pad(x, ((0, 0), (0, 13)))

    a, b = _node_proj(h, We1[:H], We1[H:2 * H], be1.reshape(1, H))

    mtile = (jnp.arange(128)[None, :] // 16
             == jnp.arange(8)[:, None]).astype(_F32)
    attrt = edge_attr.T
    zmi = jnp.zeros((N, H), _F32)
    zdx = jnp.zeros((N, 16), _F32)

    nh = NCHUNK // 2
    gathered = []
    for hx in range(2):
        sl = slice(hx * nh, (hx + 1) * nh)
        gathered.append(_sc_gather(a, b, x16, dst2d[sl], src2d[sl]))
    mis, dxs = [], []
    for hx in range(2):
        pre_a, pre_b, rel1d = gathered[hx]
        e2 = pre_a.shape[0]
        relpack = rel1d.reshape(e2 * 16 // 128, 128)
        msg, w = _edge_mlp(
            pre_a, pre_b, relpack, attrt, mtile,
            We1[2 * H:2 * H + NG], We1[2 * H + NG:],
            We2, be2.reshape(1, H), Wei, bei.reshape(1, 1),
            Wx1, bx1.reshape(1, H), Wx2, hx * (e2 // EB))
        mi2, dx2 = _sc_scatter(msg, w, rel1d, dst2d[hx * nh:(hx + 1) * nh],
                               zmi, zdx)
        mis.extend([mi2[0], mi2[1]])
        dxs.extend([dx2[0], dx2[1]])

    h_out, x_out = _node_upd(
        h, x, mask_ligand.reshape(N, 1), mis, dxs,
        Wn1[:H], Wn1[H:], bn1.reshape(1, H), Wn2, bn2.reshape(1, H))
    return (h_out, x_out)


# speedup vs baseline: 1.8023x; 1.0192x over previous
import jax
import jax.numpy as jnp
from jax import lax
from jax.experimental import pallas as pl
from jax.experimental.pallas import tpu as pltpu
from jax.experimental.pallas import tpu_sc as plsc

N = 10000
E = 320000
H = 128
EF = 4
NG = 16

NC = 2
NS = 16
NW = NC * NS

C = 100
NCHUNK = E // C
CPW = NCHUNK // NW
ROWS_PER_TILE = N // NS

EB = 6400
NB = 2000

_F32 = jnp.float32


def _silu(v):
    return v * jax.nn.sigmoid(v)


def _node_proj_body(h_ref, whi_ref, whj_ref, be1_ref, a_ref, b_ref):
    hv = h_ref[...]
    a_ref[...] = (
        jnp.dot(hv, whi_ref[...], preferred_element_type=_F32) + be1_ref[...]
    )
    b_ref[...] = jnp.dot(hv, whj_ref[...], preferred_element_type=_F32)


def _node_proj(h, whi, whj, be1):
    grid = N // NB
    return pl.pallas_call(
        _node_proj_body,
        grid=(grid,),
        in_specs=[
            pl.BlockSpec((NB, H), lambda i: (i, 0)),
            pl.BlockSpec((H, H), lambda i: (0, 0)),
            pl.BlockSpec((H, H), lambda i: (0, 0)),
            pl.BlockSpec((1, H), lambda i: (0, 0)),
        ],
        out_specs=[
            pl.BlockSpec((NB, H), lambda i: (i, 0)),
            pl.BlockSpec((NB, H), lambda i: (i, 0)),
        ],
        out_shape=[
            jax.ShapeDtypeStruct((N, H), _F32),
            jax.ShapeDtypeStruct((N, H), _F32),
        ],
    )(h, whi, whj, be1)


def _sc_gather_body(cpw, a_hbm, b_hbm, x16_hbm, dst_hbm, src_hbm,
                    pre_a_hbm, pre_b_hbm, rel_hbm,
                    idxd, idxs, buf_a, buf_b, buf_xt, buf_xs, buf_rel,
                    ga, gb, gxt, gxs, wsem, isem):
    cid = lax.axis_index("c")
    sid = lax.axis_index("s")
    wid = sid * NC + cid
    base = wid * cpw

    pltpu.sync_copy(dst_hbm.at[base], idxd[0])
    pltpu.sync_copy(src_hbm.at[base], idxs[0])

    def do_chunk(j, k):
        c = base + 2 * j + k

        @pl.when(c > base)
        def _():
            pltpu.make_async_copy(dst_hbm.at[c], idxd[k], isem[k]).wait()
            pltpu.make_async_copy(src_hbm.at[c], idxs[k], isem[k]).wait()

        @pl.when(j > 0)
        def _():
            pltpu.make_async_copy(
                buf_a[k], pre_a_hbm.at[pl.ds(c * C, C)], wsem[k]).wait()
            pltpu.make_async_copy(
                buf_b[k], pre_b_hbm.at[pl.ds(c * C, C)], wsem[k]).wait()
            pltpu.make_async_copy(
                buf_rel[k], rel_hbm.at[pl.ds(c * C * 16, C * 16)],
                wsem[k]).wait()

        @pl.when(c + 1 < base + cpw)
        def _():
            pltpu.async_copy(dst_hbm.at[c + 1], idxd[1 - k], isem[1 - k])
            pltpu.async_copy(src_hbm.at[c + 1], idxs[1 - k], isem[1 - k])

        cp_a = pltpu.async_copy(a_hbm.at[idxd[k]], buf_a[k], ga)
        cp_b = pltpu.async_copy(b_hbm.at[idxs[k]], buf_b[k], gb)
        cp_xt = pltpu.async_copy(x16_hbm.at[idxd[k]], buf_xt[k], gxt)
        cp_xs = pltpu.async_copy(x16_hbm.at[idxs[k]], buf_xs[k], gxs)
        cp_a.wait()
        cp_b.wait()
        cp_xt.wait()
        cp_xs.wait()
        for j2 in range(C):
            buf_rel[k][pl.ds(j2 * 16, 16)] = (
                buf_xt[k][j2, :] - buf_xs[k][j2, :])
        pltpu.async_copy(buf_a[k], pre_a_hbm.at[pl.ds(c * C, C)], wsem[k])
        pltpu.async_copy(buf_b[k], pre_b_hbm.at[pl.ds(c * C, C)], wsem[k])
        pltpu.async_copy(
            buf_rel[k], rel_hbm.at[pl.ds(c * C * 16, C * 16)], wsem[k])

    def body(j, carry):
        do_chunk(j, 0)
        do_chunk(j, 1)
        return carry

    lax.fori_loop(0, cpw // 2, body, 0)
    for k in range(2):
        pltpu.make_async_copy(
            buf_a[k], pre_a_hbm.at[pl.ds(base * C, C)], wsem[k]).wait()
        pltpu.make_async_copy(
            buf_b[k], pre_b_hbm.at[pl.ds(base * C, C)], wsem[k]).wait()
        pltpu.make_async_copy(
            buf_rel[k], rel_hbm.at[pl.ds(base * C * 16, C * 16)],
            wsem[k]).wait()


def _sc_gather(a, b, x16, dst2d, src2d):
    import functools
    nchunk = dst2d.shape[0]
    e2 = nchunk * C
    mesh = plsc.VectorSubcoreMesh(
        core_axis_name="c", subcore_axis_name="s",
        num_cores=NC, num_subcores=NS)
    fn = pl.kernel(
        functools.partial(_sc_gather_body, nchunk // NW),
        out_type=[
            jax.ShapeDtypeStruct((e2, H), _F32),
            jax.ShapeDtypeStruct((e2, H), _F32),
            jax.ShapeDtypeStruct((e2 * 16,), _F32),
        ],
        mesh=mesh,
        scratch_types=[
            [pltpu.VMEM((C,), jnp.int32)] * 2,
            [pltpu.VMEM((C,), jnp.int32)] * 2,
            [pltpu.VMEM((C, H), _F32)] * 2,
            [pltpu.VMEM((C, H), _F32)] * 2,
            [pltpu.VMEM((C, 16), _F32)] * 2,
            [pltpu.VMEM((C, 16), _F32)] * 2,
            [pltpu.VMEM((C * 16,), _F32)] * 2,
            pltpu.SemaphoreType.DMA,
            pltpu.SemaphoreType.DMA,
            pltpu.SemaphoreType.DMA,
            pltpu.SemaphoreType.DMA,
            [pltpu.SemaphoreType.DMA] * 2,
            [pltpu.SemaphoreType.DMA] * 2,
        ],
        compiler_params=pltpu.CompilerParams(use_tc_tiling_on_sc=False),
    )
    return fn(a, b, x16, dst2d, src2d)


def _edge_mlp_body(pre_a_ref, pre_b_ref, relp_ref, attrt_ref, mtile_ref,
                   wd_ref, wa_ref, we2_ref, be2_ref, wei_ref,
                   bei_ref, wx1_ref, bx1_ref, wx2_ref,
                   msg_ref, w_ref):
    pre = pre_a_ref[...] + pre_b_ref[...]
    relp = relp_ref[...]
    rep = jnp.broadcast_to(
        relp[:, None, :], (EB // 8, 8, 128)).reshape(EB, 128)
    m = jnp.broadcast_to(
        mtile_ref[...][None], (EB // 8, 8, 128)).reshape(EB, 128)
    rel_ext = rep * m
    ones_col = jnp.ones((128, 1), _F32)
    d_sq = jnp.dot(rel_ext * rel_ext, ones_col,
                   preferred_element_type=_F32)
    d16 = jnp.sqrt(jnp.broadcast_to(d_sq, (EB, 16)) + 1e-08)
    step = 10.0 / (NG - 1)
    offs = lax.broadcasted_iota(jnp.int32, (1, NG), 1).astype(_F32) * step
    coeff = -0.5 / step**2
    d_feat = jnp.exp(coeff * (d16 - offs) ** 2)
    pre = pre + jnp.dot(d_feat, wd_ref[...], preferred_element_type=_F32)
    pre = pre + lax.dot_general(
        attrt_ref[...], wa_ref[...],
        dimension_numbers=(((0,), (0,)), ((), ())),
        preferred_element_type=_F32)
    t = _silu(pre)
    mij = _silu(
        jnp.dot(t, we2_ref[...], preferred_element_type=_F32) + be2_ref[...])
    e16 = jnp.broadcast_to(
        jnp.dot(mij, wei_ref[...], preferred_element_type=_F32)
        + bei_ref[...], (EB, 16))
    eij = jax.nn.sigmoid(e16)[:, :1]
    msg_ref[...] = mij * eij
    g = _silu(
        jnp.dot(mij, wx1_ref[...], preferred_element_type=_F32) + bx1_ref[...])
    s16 = jnp.broadcast_to(
        jnp.dot(g, wx2_ref[...], preferred_element_type=_F32), (EB, 16))
    w_ref[...] = jnp.tanh(s16) / (d16 + 1.0)


def _edge_mlp(pre_a, pre_b, relpack, attrt, mtile,
              wd, wa, we2, be2, wei, bei, wx1, bx1, wx2, off):
    e2 = pre_a.shape[0]
    grid = e2 // EB
    full = lambda r, c: pl.BlockSpec((r, c), lambda i: (0, 0))
    return pl.pallas_call(
        _edge_mlp_body,
        grid=(grid,),
        in_specs=[
            pl.BlockSpec((EB, H), lambda i: (i, 0)),
            pl.BlockSpec((EB, H), lambda i: (i, 0)),
            pl.BlockSpec((EB * 16 // 128, 128), lambda i: (i, 0)),
            pl.BlockSpec((EF, EB), lambda i: (0, i + off)),
            full(8, 128),
            full(NG, H), full(EF, H), full(H, H), full(1, H),
            full(H, 1), full(1, 1), full(H, H), full(1, H), full(H, 1),
        ],
        out_specs=[
            pl.BlockSpec((EB, H), lambda i: (i, 0)),
            pl.BlockSpec((EB, 16), lambda i: (i, 0)),
        ],
        out_shape=[
            jax.ShapeDtypeStruct((e2, H), _F32),
            jax.ShapeDtypeStruct((e2, 16), _F32),
        ],
        compiler_params=pltpu.CompilerParams(
            vmem_limit_bytes=50 * 1024 * 1024),
    )(pre_a, pre_b, relpack, attrt, mtile,
      wd, wa, we2, be2, wei, bei, wx1, bx1, wx2)


def _sc_scatter_body(cpw, msg_hbm, w_hbm, rel_hbm, dst_hbm, zmi_hbm, zdx_hbm,
                     mi2_hbm, dx2_hbm,
                     mbuf, wbuf, rbuf, dbuf, idxd, acc_mi, acc_dx,
                     lm, lw, lr, asem):
    cid = lax.axis_index("c")
    sid = lax.axis_index("s")
    wid = sid * NC + cid
    base = wid * cpw
    r0 = sid * ROWS_PER_TILE

    pltpu.sync_copy(zmi_hbm.at[pl.ds(r0, ROWS_PER_TILE)],
                    acc_mi.at[pl.ds(r0, ROWS_PER_TILE)])
    pltpu.sync_copy(zdx_hbm.at[pl.ds(r0, ROWS_PER_TILE)],
                    acc_dx.at[pl.ds(r0, ROWS_PER_TILE)])
    plsc.subcore_barrier()

    def do_chunk(j, k):
        c = base + 2 * j + k

        @pl.when(j > 0)
        def _():
            pltpu.make_async_copy(mbuf[k], acc_mi.at[idxd[k]], asem[k]).wait()
            pltpu.make_async_copy(dbuf[k], acc_dx.at[idxd[k]], asem[k]).wait()

        pltpu.sync_copy(dst_hbm.at[c], idxd[k])
        cp_m = pltpu.async_copy(msg_hbm.at[pl.ds(c * C, C)], mbuf[k], lm)
        cp_w = pltpu.async_copy(w_hbm.at[pl.ds(c * C, C)], wbuf[k], lw)
        cp_r = pltpu.async_copy(
            rel_hbm.at[pl.ds(c * C * 16, C * 16)], rbuf[k], lr)
        cp_m.wait()
        cp_w.wait()
        cp_r.wait()
        for j2 in range(C):
            dbuf[k][j2, :] = rbuf[k][pl.ds(j2 * 16, 16)] * wbuf[k][j2, :]
        pltpu.async_copy(mbuf[k], acc_mi.at[idxd[k]], asem[k], add=True)
        pltpu.async_copy(dbuf[k], acc_dx.at[idxd[k]], asem[k], add=True)

    def body(j, carry):
        do_chunk(j, 0)
        do_chunk(j, 1)
        return carry

    lax.fori_loop(0, cpw // 2, body, 0)
    for k in range(2):
        pltpu.make_async_copy(mbuf[k], acc_mi.at[idxd[k]], asem[k]).wait()
        pltpu.make_async_copy(dbuf[k], acc_dx.at[idxd[k]], asem[k]).wait()
    plsc.subcore_barrier()

    pltpu.sync_copy(acc_mi.at[pl.ds(r0, ROWS_PER_TILE)],
                    mi2_hbm.at[cid, pl.ds(r0, ROWS_PER_TILE)])
    pltpu.sync_copy(acc_dx.at[pl.ds(r0, ROWS_PER_TILE)],
                    dx2_hbm.at[cid, pl.ds(r0, ROWS_PER_TILE)])


def _sc_scatter(msg, w, rel1d, dst2d, zmi, zdx):
    import functools
    nchunk = dst2d.shape[0]
    mesh = plsc.VectorSubcoreMesh(
        core_axis_name="c", subcore_axis_name="s",
        num_cores=NC, num_subcores=NS)
    fn = pl.kernel(
        functools.partial(_sc_scatter_body, nchunk // NW),
        out_type=[
            jax.ShapeDtypeStruct((NC, N, H), _F32),
            jax.ShapeDtypeStruct((NC, N, 16), _F32),
        ],
        mesh=mesh,
        scratch_types=[
            [pltpu.VMEM((C, H), _F32)] * 2,
            [pltpu.VMEM((C, 16), _F32)] * 2,
            [pltpu.VMEM((C * 16,), _F32)] * 2,
            [pltpu.VMEM((C, 16), _F32)] * 2,
            [pltpu.VMEM((C,), jnp.int32)] * 2,
            pltpu.VMEM_SHARED((N, H), _F32),
            pltpu.VMEM_SHARED((N, 16), _F32),
            pltpu.SemaphoreType.DMA,
            pltpu.SemaphoreType.DMA,
            pltpu.SemaphoreType.DMA,
            [pltpu.SemaphoreType.DMA] * 2,
        ],
        compiler_params=pltpu.CompilerParams(use_tc_tiling_on_sc=False),
    )
    return fn(msg, w, rel1d, dst2d, zmi, zdx)


def _node_upd_body(h_ref, x_ref, mask_ref, mi0_ref, mi1_ref, mi2_ref, mi3_ref,
                   dx0_ref, dx1_ref, dx2_ref, dx3_ref,
                   wm_ref, wh_ref, bn1_ref, wn2_ref, bn2_ref,
                   ho_ref, xo_ref):
    hv = h_ref[...]
    mi = (mi0_ref[...] + mi1_ref[...]) + (mi2_ref[...] + mi3_ref[...])
    t = _silu(
        jnp.dot(mi, wm_ref[...], preferred_element_type=_F32)
        + jnp.dot(hv, wh_ref[...], preferred_element_type=_F32)
        + bn1_ref[...])
    dh = jnp.dot(t, wn2_ref[...], preferred_element_type=_F32) + bn2_ref[...]
    ho_ref[...] = hv + dh
    dx = (dx0_ref[...] + dx1_ref[...]) + (dx2_ref[...] + dx3_ref[...])
    xo_ref[...] = x_ref[...] + dx[:, :3] * mask_ref[...]


def _node_upd(h, x, mask, mis, dxs, wm, wh, bn1, wn2, bn2):
    grid = N // NB
    full = lambda r, c: pl.BlockSpec((r, c), lambda i: (0, 0))
    return pl.pallas_call(
        _node_upd_body,
        grid=(grid,),
        in_specs=[
            pl.BlockSpec((NB, H), lambda i: (i, 0)),
            pl.BlockSpec((NB, 3), lambda i: (i, 0)),
            pl.BlockSpec((NB, 1), lambda i: (i, 0)),
            pl.BlockSpec((NB, H), lambda i: (i, 0)),
            pl.BlockSpec((NB, H), lambda i: (i, 0)),
            pl.BlockSpec((NB, H), lambda i: (i, 0)),
            pl.BlockSpec((NB, H), lambda i: (i, 0)),
            pl.BlockSpec((NB, 16), lambda i: (i, 0)),
            pl.BlockSpec((NB, 16), lambda i: (i, 0)),
            pl.BlockSpec((NB, 16), lambda i: (i, 0)),
            pl.BlockSpec((NB, 16), lambda i: (i, 0)),
            full(H, H), full(H, H), full(1, H), full(H, H), full(1, H),
        ],
        out_specs=[
            pl.BlockSpec((NB, H), lambda i: (i, 0)),
            pl.BlockSpec((NB, 3), lambda i: (i, 0)),
        ],
        out_shape=[
            jax.ShapeDtypeStruct((N, H), _F32),
            jax.ShapeDtypeStruct((N, 3), _F32),
        ],
    )(h, x, mask, *mis, *dxs, wm, wh, bn1, wn2, bn2)


def kernel(h, x, edge_index, mask_ligand, edge_attr,
           We1, be1, We2, be2, Wei, bei, Wx1, bx1, Wx2,
           Wn1, bn1, Wn2, bn2):
    src = edge_index[0]
    dst = edge_index[1]
    dst2d = dst.reshape(NCHUNK, C)
    src2d = src.reshape(NCHUNK, C)
    x16 = jnp.pad(x, ((0, 0), (0, 13)))

    a, b = _node_proj(h, We1[:H], We1[H:2 * H], be1.reshape(1, H))

    mtile = (jnp.arange(128)[None, :] // 16
             == jnp.arange(8)[:, None]).astype(_F32)
    attrt = edge_attr.T
    zmi = jnp.zeros((N, H), _F32)
    zdx = jnp.zeros((N, 16), _F32)

    nh = NCHUNK // 2
    gathered = []
    for hx in range(2):
        sl = slice(hx * nh, (hx + 1) * nh)
        gathered.append(_sc_gather(a, b, x16, dst2d[sl], src2d[sl]))
    mis, dxs = [], []
    for hx in range(2):
        pre_a, pre_b, rel1d = gathered[hx]
        e2 = pre_a.shape[0]
        relpack = rel1d.reshape(e2 * 16 // 128, 128)
        msg, w = _edge_mlp(
            pre_a, pre_b, relpack, attrt, mtile,
            We1[2 * H:2 * H + NG], We1[2 * H + NG:],
            We2, be2.reshape(1, H), Wei, bei.reshape(1, 1),
            Wx1, bx1.reshape(1, H), Wx2, hx * (e2 // EB))
        mi2, dx2 = _sc_scatter(msg, w, rel1d, dst2d[hx * nh:(hx + 1) * nh],
                               zmi, zdx)
        mis.extend([mi2[0], mi2[1]])
        dxs.extend([dx2[0], dx2[1]])

    h_out, x_out = _node_upd(
        h, x, mask_ligand.reshape(N, 1), mis, dxs,
        Wn1[:H], Wn1[H:], bn1.reshape(1, H), Wn2, bn2.reshape(1, H))
    return (h_out, x_out)
